# baseline (speedup 1.0000x reference)
import functools

import jax
import jax.numpy as jnp
from jax import lax
from jax.experimental import pallas as pl
from jax.experimental.pallas import tpu as pltpu
from jax.experimental.pallas import tpu_sc as plsc

SEQ = 8192
EMB = 64

_cached = None


def _build():
    global _cached
    if _cached is not None:
        return _cached

    info = plsc.get_sparse_core_info()
    nc, ns = info.num_cores, info.num_subcores
    nw = nc * ns
    bpw = SEQ // nw
    mesh = plsc.VectorSubcoreMesh(core_axis_name="c", subcore_axis_name="s")

    @functools.partial(
        pl.kernel,
        mesh=mesh,
        out_type=jax.ShapeDtypeStruct((SEQ, EMB), jnp.float32),
        scratch_types=[
            pltpu.VMEM((bpw,), jnp.int32),
            pltpu.VMEM((bpw, EMB), jnp.float32),
            pltpu.VMEM((bpw, EMB), jnp.float32),
            pltpu.SemaphoreType.DMA,
        ],
        compiler_params=pltpu.CompilerParams(use_tc_tiling_on_sc=False),
    )
    def k(x_hbm, tok_hbm, pos_hbm, out_hbm, idx_v, rows_v, pos_v, sem):
        wid = lax.axis_index("s") * nc + lax.axis_index("c")
        base = wid * bpw
        pltpu.sync_copy(x_hbm.at[pl.ds(base, bpw)], idx_v)
        gather = pltpu.async_copy(tok_hbm.at[idx_v], rows_v, sem)
        pltpu.sync_copy(pos_hbm.at[pl.ds(SEQ - base - bpw, bpw)], pos_v)
        gather.wait()

        def body(j, carry):
            rj = bpw - 1 - j
            for c in range(EMB // 16):
                sl = pl.ds(c * 16, 16)
                rows_v[j, sl] = rows_v[j, sl] + pos_v[rj, sl]
            return carry

        lax.fori_loop(0, bpw, body, 0)
        pltpu.sync_copy(rows_v, out_hbm.at[pl.ds(base, bpw)])

    _cached = k
    return _cached


def kernel(x, token_table, pos_table):
    return _build()(x.astype(jnp.int32), token_table, pos_table)


# speedup vs baseline: 1.6564x; 1.6564x over previous
import functools

import jax
import jax.numpy as jnp
from jax import lax
from jax.experimental import pallas as pl
from jax.experimental.pallas import tpu as pltpu
from jax.experimental.pallas import tpu_sc as plsc

SEQ = 8192
EMB = 64
FIRE = 16

_cached = None


def _build():
    global _cached
    if _cached is not None:
        return _cached

    info = plsc.get_sparse_core_info()
    nc, ns = info.num_cores, info.num_subcores
    nw = nc * ns
    bpw = SEQ // nw
    mesh = plsc.VectorSubcoreMesh(core_axis_name="c", subcore_axis_name="s")

    @functools.partial(
        pl.kernel,
        mesh=mesh,
        out_type=jax.ShapeDtypeStruct((SEQ, EMB), jnp.float32),
        scratch_types=[
            pltpu.VMEM((bpw,), jnp.int32),
            pltpu.VMEM((bpw, EMB), jnp.float32),
            pltpu.VMEM((bpw, EMB), jnp.float32),
            pltpu.SemaphoreType.DMA,
            pltpu.SemaphoreType.DMA,
        ],
    )
    def k(x_hbm, tok_hbm, pos_hbm, out_hbm, idx_v, rows_v, pos_v, sem, gsem):
        wid = lax.axis_index("s") * nc + lax.axis_index("c")
        base = wid * bpw
        pltpu.sync_copy(x_hbm.at[pl.ds(base, bpw)], idx_v)
        pcp = pltpu.async_copy(
            pos_hbm.at[pl.ds(SEQ - base - bpw, bpw)], pos_v, sem
        )

        def fetch(g, carry):
            jj = g * FIRE
            vec = idx_v[pl.ds(jj, FIRE)]
            cps = []
            for b in range(FIRE):
                r = vec[b]
                cps.append(
                    pltpu.async_copy(
                        tok_hbm.at[pl.ds(r, 1)],
                        rows_v.at[pl.ds(jj + b, 1)],
                        gsem,
                    )
                )
            for cp in cps:
                cp.wait()
            return carry

        lax.fori_loop(0, bpw // FIRE, fetch, 0)
        pcp.wait()

        def body(j, carry):
            rj = bpw - 1 - j
            for c in range(EMB // 16):
                sl = pl.ds(c * 16, 16)
                rows_v[j, sl] = rows_v[j, sl] + pos_v[rj, sl]
            return carry

        lax.fori_loop(0, bpw, body, 0)
        pltpu.sync_copy(rows_v, out_hbm.at[pl.ds(base, bpw)])

    _cached = k
    return _cached


def kernel(x, token_table, pos_table):
    return _build()(x.astype(jnp.int32), token_table, pos_table)
